# Initial kernel scaffold; baseline (speedup 1.0000x reference)
import jax
import jax.numpy as jnp
from jax.experimental import pallas as pl


def kernel(x, edge_index, W1, b1, g1, be1, W2, b2, g2, be2, Wc, bc):
    raise NotImplementedError("write your pallas kernel here")


# speedup vs baseline: 21.6903x; 21.6903x over previous
import functools

import jax
import jax.numpy as jnp
from jax import lax
from jax.experimental import pallas as pl
from jax.experimental.pallas import tpu as pltpu
from jax.experimental.pallas import tpu_sc as plsc

N = 10000
E = 320000
D = 128
H = 128
OUT = 70

NC = 2
NS = 16
NW = NC * NS

CH = 128
EPW = 10240
NCHUNK = EPW // CH
EPAD = NW * EPW - E

PADROWS = 240
NR = N + PADROWS
RPT = NR // NS
ZR = 80
NZ = RPT // ZR

_PREC = jax.lax.Precision.HIGHEST


def _deg_body(dst_hbm, out_hbm, idst, ones_v, zbuf, acc):
    c = lax.axis_index("c")
    s = lax.axis_index("s")
    wid = c * NS + s
    base = s * RPT

    def fill(i, carry):
        ones_v[i, :] = jnp.full((16,), 1.0, jnp.float32)
        return carry

    lax.fori_loop(0, CH, fill, 0)

    def zfill(i, carry):
        zbuf[i, :] = jnp.zeros((16,), jnp.float32)
        return carry

    lax.fori_loop(0, ZR, zfill, 0)

    def zero(k, carry):
        pltpu.sync_copy(zbuf, acc.at[pl.ds(base + k * ZR, ZR)])
        return carry

    lax.fori_loop(0, NZ, zero, 0)

    pltpu.sync_copy(dst_hbm.at[wid], idst)
    plsc.subcore_barrier()

    def body(j, carry):
        pltpu.sync_copy(ones_v, acc.at[idst.at[j]], add=True)
        return carry

    lax.fori_loop(0, NCHUNK, body, 0)
    plsc.subcore_barrier()

    def copy_out(k, carry):
        r0 = base + k * ZR
        pltpu.sync_copy(acc.at[pl.ds(r0, ZR)], out_hbm.at[c, pl.ds(r0, ZR)])
        return carry

    lax.fori_loop(0, NZ, copy_out, 0)


def _agg_body(xs_hbm, src_hbm, dst_hbm, out_hbm, isrc, idst, rows, zbuf, acc, sem):
    c = lax.axis_index("c")
    s = lax.axis_index("s")
    wid = c * NS + s
    base = s * RPT

    def zfill(i, carry):
        for g in range(H // 16):
            zbuf[i, pl.ds(16 * g, 16)] = jnp.zeros((16,), jnp.float32)
        return carry

    lax.fori_loop(0, ZR, zfill, 0)

    def zero(k, carry):
        pltpu.sync_copy(zbuf, acc.at[pl.ds(base + k * ZR, ZR)])
        return carry

    lax.fori_loop(0, NZ, zero, 0)

    pltpu.sync_copy(src_hbm.at[wid], isrc)
    pltpu.sync_copy(dst_hbm.at[wid], idst)
    plsc.subcore_barrier()

    def body(j, carry):
        pltpu.async_copy(xs_hbm.at[isrc.at[j]], rows, sem).wait()
        pltpu.sync_copy(rows, acc.at[idst.at[j]], add=True)
        return carry

    lax.fori_loop(0, NCHUNK, body, 0)
    plsc.subcore_barrier()

    def copy_out(k, carry):
        r0 = base + k * ZR
        pltpu.sync_copy(acc.at[pl.ds(r0, ZR)], out_hbm.at[c, pl.ds(r0, ZR)])
        return carry

    lax.fori_loop(0, NZ, copy_out, 0)


@functools.lru_cache(maxsize=None)
def _sc_kernels():
    mesh = plsc.VectorSubcoreMesh(core_axis_name="c", subcore_axis_name="s",
                                  num_cores=NC, num_subcores=NS)
    deg = pl.kernel(
        _deg_body,
        out_type=jax.ShapeDtypeStruct((NC, NR, 16), jnp.float32),
        mesh=mesh,
        compiler_params=pltpu.CompilerParams(use_tc_tiling_on_sc=False),
        scratch_types=[
            pltpu.VMEM((NCHUNK, CH), jnp.int32),
            pltpu.VMEM((CH, 16), jnp.float32),
            pltpu.VMEM((ZR, 16), jnp.float32),
            pltpu.VMEM_SHARED((NR, 16), jnp.float32),
        ],
    )
    agg = pl.kernel(
        _agg_body,
        out_type=jax.ShapeDtypeStruct((NC, NR, H), jnp.float32),
        mesh=mesh,
        scratch_types=[
            pltpu.VMEM((NCHUNK, CH), jnp.int32),
            pltpu.VMEM((NCHUNK, CH), jnp.int32),
            pltpu.VMEM((CH, H), jnp.float32),
            pltpu.VMEM((ZR, H), jnp.float32),
            pltpu.VMEM_SHARED((NR, H), jnp.float32),
            pltpu.SemaphoreType.DMA,
        ],
    )
    return deg, agg


def _tc1_body(x_ref, w1_ref, dpart_ref, xs_ref, dinv_ref):
    deg = dpart_ref[0, :N, :] + dpart_ref[1, :N, :] + 1.0
    dinv16 = lax.rsqrt(jnp.maximum(deg, 1.0))
    dinv = dinv16[:, 0:1]
    xw = jnp.dot(x_ref[...], w1_ref[...], preferred_element_type=jnp.float32,
                 precision=_PREC)
    xs_ref[...] = xw * dinv
    dinv_ref[...] = dinv


def _tc2_body(a_ref, xs_ref, dinv_ref, b_ref, g_ref, be_ref, w_ref, out_ref):
    dinv = dinv_ref[...]
    h = (a_ref[0, :N, :] + a_ref[1, :N, :] + xs_ref[...]) * dinv + b_ref[...]
    mu = jnp.mean(h, axis=0, keepdims=True)
    var = jnp.mean((h - mu) ** 2, axis=0, keepdims=True)
    h = (h - mu) * lax.rsqrt(var + 1e-5) * g_ref[...] + be_ref[...]
    h = jnp.maximum(h, 0.0)
    out_ref[...] = jnp.dot(h, w_ref[...], preferred_element_type=jnp.float32,
                           precision=_PREC) * dinv


def _tc3_body(a_ref, xs_ref, dinv_ref, b_ref, g_ref, be_ref, wc_ref, bc_ref, out_ref):
    h = (a_ref[0, :N, :] + a_ref[1, :N, :] + xs_ref[...]) * dinv_ref[...] + b_ref[...]
    mu = jnp.mean(h, axis=0, keepdims=True)
    var = jnp.mean((h - mu) ** 2, axis=0, keepdims=True)
    h = (h - mu) * lax.rsqrt(var + 1e-5) * g_ref[...] + be_ref[...]
    out_ref[...] = jnp.dot(h, wc_ref[...], preferred_element_type=jnp.float32,
                           precision=_PREC) + bc_ref[...]


_tc1 = pl.pallas_call(
    _tc1_body,
    out_shape=[jax.ShapeDtypeStruct((N, H), jnp.float32),
               jax.ShapeDtypeStruct((N, 1), jnp.float32)],
)

_tc2 = pl.pallas_call(
    _tc2_body,
    out_shape=jax.ShapeDtypeStruct((N, H), jnp.float32),
)

_tc3 = pl.pallas_call(
    _tc3_body,
    out_shape=jax.ShapeDtypeStruct((N, OUT), jnp.float32),
)


def kernel(x, edge_index, W1, b1, g1, be1, W2, b2, g2, be2, Wc, bc):
    src = edge_index[0].astype(jnp.int32)
    dst = edge_index[1].astype(jnp.int32)
    pad_iota = jnp.arange(EPAD, dtype=jnp.int32)
    src3 = jnp.concatenate([src, pad_iota % N]).reshape(NW, NCHUNK, CH)
    dst3 = jnp.concatenate([dst, N + (pad_iota % PADROWS)]).reshape(NW, NCHUNK, CH)

    deg_kernel, agg_kernel = _sc_kernels()
    dparts = deg_kernel(dst3)
    xs1, dinv = _tc1(x, W1, dparts)
    agg1 = agg_kernel(xs1, src3, dst3)
    xs2 = _tc2(agg1, xs1, dinv, b1, g1, be1, W2)
    agg2 = agg_kernel(xs2, src3, dst3)
    out = _tc3(agg2, xs2, dinv, b2, g2, be2, Wc, bc)
    return out


# speedup vs baseline: 30.8670x; 1.4231x over previous
import functools

import jax
import jax.numpy as jnp
from jax import lax
from jax.experimental import pallas as pl
from jax.experimental.pallas import tpu as pltpu
from jax.experimental.pallas import tpu_sc as plsc

N = 10000
E = 320000
D = 128
H = 128
OUT = 70

NC = 2
NS = 16
NW = NC * NS

CH = 128
EPW = 10240
NCHUNK = EPW // CH
BLK = 8
NBLK = NCHUNK // BLK
EPAD = NW * EPW - E

PADROWS = 240
NR = N + PADROWS
RPT = NR // NS
ZR = 80
NZ = RPT // ZR

_PREC = jax.lax.Precision.HIGHEST


def _deg_body(dst_hbm, out_hbm, idst, ones_v, zbuf, acc):
    c = lax.axis_index("c")
    s = lax.axis_index("s")
    wid = c * NS + s
    base = s * RPT

    def fill(i, carry):
        ones_v[i, :] = jnp.full((16,), 1.0, jnp.float32)
        return carry

    lax.fori_loop(0, CH, fill, 0)

    def zfill(i, carry):
        zbuf[i, :] = jnp.zeros((16,), jnp.float32)
        return carry

    lax.fori_loop(0, ZR, zfill, 0)

    def zero(k, carry):
        pltpu.sync_copy(zbuf, acc.at[pl.ds(base + k * ZR, ZR)])
        return carry

    lax.fori_loop(0, NZ, zero, 0)

    pltpu.sync_copy(dst_hbm.at[wid], idst)
    plsc.subcore_barrier()

    def body(j, carry):
        pltpu.sync_copy(ones_v, acc.at[idst.at[j]], add=True)
        return carry

    lax.fori_loop(0, NCHUNK, body, 0)
    plsc.subcore_barrier()

    def copy_out(k, carry):
        r0 = base + k * ZR
        pltpu.sync_copy(acc.at[pl.ds(r0, ZR)], out_hbm.at[c, pl.ds(r0, ZR)])
        return carry

    lax.fori_loop(0, NZ, copy_out, 0)


def _agg_body(xs_hbm, src_hbm, dst_hbm, out_hbm, isrc0, isrc1, idst, rows0,
              rows1, acc, sem0, sem1, semi):
    c = lax.axis_index("c")
    s = lax.axis_index("s")
    wid = c * NS + s
    base = s * RPT

    def zfill(i, carry):
        for g in range(H // 16):
            rows0[i, pl.ds(16 * g, 16)] = jnp.zeros((16,), jnp.float32)
        return carry

    lax.fori_loop(0, CH, zfill, 0)
    for k in range(RPT // CH):
        pltpu.sync_copy(rows0, acc.at[pl.ds(base + k * CH, CH)])

    pltpu.sync_copy(dst_hbm.at[wid], idst)
    pltpu.sync_copy(src_hbm.at[wid, pl.ds(0, BLK)], isrc0)
    plsc.subcore_barrier()

    pltpu.async_copy(xs_hbm.at[isrc0.at[0]], rows0, sem0)

    def half_block(j0, cur, nxt, nxt_blk):
        pltpu.async_copy(src_hbm.at[wid, pl.ds(nxt_blk * BLK, BLK)], nxt, semi)
        for p in range(BLK // 2):
            a, b = 2 * p, 2 * p + 1
            pltpu.async_copy(xs_hbm.at[cur.at[b]], rows1, sem1)
            pltpu.make_async_copy(xs_hbm.at[cur.at[a]], rows0, sem0).wait()
            pltpu.sync_copy(rows0, acc.at[idst.at[j0 + a]], add=True)
            if b + 1 < BLK:
                pltpu.async_copy(xs_hbm.at[cur.at[b + 1]], rows0, sem0)
            else:
                pltpu.make_async_copy(
                    src_hbm.at[wid, pl.ds(0, BLK)], nxt, semi).wait()
                pltpu.async_copy(xs_hbm.at[nxt.at[0]], rows0, sem0)
            pltpu.make_async_copy(xs_hbm.at[cur.at[b]], rows1, sem1).wait()
            pltpu.sync_copy(rows1, acc.at[idst.at[j0 + b]], add=True)

    def body(t, carry):
        b0 = 2 * t
        half_block(b0 * BLK, isrc0, isrc1, b0 + 1)
        half_block(b0 * BLK + BLK, isrc1, isrc0, (b0 + 2) % NBLK)
        return carry

    lax.fori_loop(0, NBLK // 2, body, 0)
    pltpu.make_async_copy(xs_hbm.at[isrc0.at[0]], rows0, sem0).wait()
    plsc.subcore_barrier()

    def copy_out(k, carry):
        r0 = base + k * ZR
        pltpu.sync_copy(acc.at[pl.ds(r0, ZR)], out_hbm.at[c, pl.ds(r0, ZR)])
        return carry

    lax.fori_loop(0, NZ, copy_out, 0)


@functools.lru_cache(maxsize=None)
def _sc_kernels():
    mesh = plsc.VectorSubcoreMesh(core_axis_name="c", subcore_axis_name="s",
                                  num_cores=NC, num_subcores=NS)
    deg = pl.kernel(
        _deg_body,
        out_type=jax.ShapeDtypeStruct((NC, NR, 16), jnp.float32),
        mesh=mesh,
        compiler_params=pltpu.CompilerParams(use_tc_tiling_on_sc=False),
        scratch_types=[
            pltpu.VMEM((NCHUNK, CH), jnp.int32),
            pltpu.VMEM((CH, 16), jnp.float32),
            pltpu.VMEM((ZR, 16), jnp.float32),
            pltpu.VMEM_SHARED((NR, 16), jnp.float32),
        ],
    )
    agg = pl.kernel(
        _agg_body,
        out_type=jax.ShapeDtypeStruct((NC, NR, H), jnp.float32),
        mesh=mesh,
        scratch_types=[
            pltpu.VMEM((BLK, CH), jnp.int32),
            pltpu.VMEM((BLK, CH), jnp.int32),
            pltpu.VMEM((NCHUNK, CH), jnp.int32),
            pltpu.VMEM((CH, H), jnp.float32),
            pltpu.VMEM((CH, H), jnp.float32),
            pltpu.VMEM_SHARED((NR, H), jnp.float32),
            pltpu.SemaphoreType.DMA,
            pltpu.SemaphoreType.DMA,
            pltpu.SemaphoreType.DMA,
        ],
    )
    return deg, agg


def _tc1_body(x_ref, w1_ref, dpart_ref, xs_ref, dinv_ref):
    deg = dpart_ref[0, :N, :] + dpart_ref[1, :N, :] + 1.0
    dinv16 = lax.rsqrt(jnp.maximum(deg, 1.0))
    dinv = dinv16[:, 0:1]
    xw = jnp.dot(x_ref[...], w1_ref[...], preferred_element_type=jnp.float32,
                 precision=_PREC)
    xs_ref[...] = xw * dinv
    dinv_ref[...] = dinv


def _tc2_body(a_ref, xs_ref, dinv_ref, b_ref, g_ref, be_ref, w_ref, out_ref):
    dinv = dinv_ref[...]
    h = (a_ref[0, :N, :] + a_ref[1, :N, :] + xs_ref[...]) * dinv + b_ref[...]
    mu = jnp.mean(h, axis=0, keepdims=True)
    var = jnp.mean((h - mu) ** 2, axis=0, keepdims=True)
    h = (h - mu) * lax.rsqrt(var + 1e-5) * g_ref[...] + be_ref[...]
    h = jnp.maximum(h, 0.0)
    out_ref[...] = jnp.dot(h, w_ref[...], preferred_element_type=jnp.float32,
                           precision=_PREC) * dinv


def _tc3_body(a_ref, xs_ref, dinv_ref, b_ref, g_ref, be_ref, wc_ref, bc_ref, out_ref):
    h = (a_ref[0, :N, :] + a_ref[1, :N, :] + xs_ref[...]) * dinv_ref[...] + b_ref[...]
    mu = jnp.mean(h, axis=0, keepdims=True)
    var = jnp.mean((h - mu) ** 2, axis=0, keepdims=True)
    h = (h - mu) * lax.rsqrt(var + 1e-5) * g_ref[...] + be_ref[...]
    out_ref[...] = jnp.dot(h, wc_ref[...], preferred_element_type=jnp.float32,
                           precision=_PREC) + bc_ref[...]


_tc1 = pl.pallas_call(
    _tc1_body,
    out_shape=[jax.ShapeDtypeStruct((N, H), jnp.float32),
               jax.ShapeDtypeStruct((N, 1), jnp.float32)],
)

_tc2 = pl.pallas_call(
    _tc2_body,
    out_shape=jax.ShapeDtypeStruct((N, H), jnp.float32),
)

_tc3 = pl.pallas_call(
    _tc3_body,
    out_shape=jax.ShapeDtypeStruct((N, OUT), jnp.float32),
)


def kernel(x, edge_index, W1, b1, g1, be1, W2, b2, g2, be2, Wc, bc):
    src = edge_index[0].astype(jnp.int32)
    dst = edge_index[1].astype(jnp.int32)
    pad_iota = jnp.arange(EPAD, dtype=jnp.int32)
    src3 = jnp.concatenate([src, pad_iota % N]).reshape(NW, NCHUNK, CH)
    dst3 = jnp.concatenate([dst, N + (pad_iota % PADROWS)]).reshape(NW, NCHUNK, CH)

    deg_kernel, agg_kernel = _sc_kernels()
    dparts = deg_kernel(dst3)
    xs1, dinv = _tc1(x, W1, dparts)
    agg1 = agg_kernel(xs1, src3, dst3)
    xs2 = _tc2(agg1, xs1, dinv, b1, g1, be1, W2)
    agg2 = agg_kernel(xs2, src3, dst3)
    out = _tc3(agg2, xs2, dinv, b2, g2, be2, Wc, bc)
    return out


# speedup vs baseline: 30.9752x; 1.0035x over previous
import functools

import jax
import jax.numpy as jnp
from jax import lax
from jax.experimental import pallas as pl
from jax.experimental.pallas import tpu as pltpu
from jax.experimental.pallas import tpu_sc as plsc

N = 10000
E = 320000
D = 128
H = 128
OUT = 70

NC = 2
NS = 16
NW = NC * NS

CH = 128
EPW = 10240
NCHUNK = EPW // CH
BLK = 8
NBLK = NCHUNK // BLK
EPAD = NW * EPW - E

PADROWS = 240
NR = N + PADROWS
RPT = NR // NS
ZR = 80
NZ = RPT // ZR

_PREC = jax.lax.Precision.HIGHEST


def _deg_body(dst_hbm, out_hbm, idst, ones_v, zbuf, acc):
    c = lax.axis_index("c")
    s = lax.axis_index("s")
    wid = c * NS + s
    base = s * RPT

    def fill(i, carry):
        ones_v[i, :] = jnp.full((16,), 1.0, jnp.float32)
        return carry

    lax.fori_loop(0, CH, fill, 0)

    def zfill(i, carry):
        zbuf[i, :] = jnp.zeros((16,), jnp.float32)
        return carry

    lax.fori_loop(0, ZR, zfill, 0)

    def zero(k, carry):
        pltpu.sync_copy(zbuf, acc.at[pl.ds(base + k * ZR, ZR)])
        return carry

    lax.fori_loop(0, NZ, zero, 0)

    pltpu.sync_copy(dst_hbm.at[wid], idst)
    plsc.subcore_barrier()

    def body(j, carry):
        pltpu.sync_copy(ones_v, acc.at[idst.at[j]], add=True)
        return carry

    lax.fori_loop(0, NCHUNK, body, 0)
    plsc.subcore_barrier()

    def copy_out(k, carry):
        r0 = base + k * ZR
        pltpu.sync_copy(acc.at[pl.ds(r0, ZR)], out_hbm.at[c, pl.ds(r0, ZR)])
        return carry

    lax.fori_loop(0, NZ, copy_out, 0)


def _agg_body(xs_hbm, src_hbm, dst_hbm, out_hbm, isrc0, isrc1, idst, rows0,
              rows1, acc, sem0, sem1, semi):
    c = lax.axis_index("c")
    s = lax.axis_index("s")
    wid = c * NS + s
    base = s * RPT

    def zfill(i, carry):
        for g in range(H // 16):
            rows0[i, pl.ds(16 * g, 16)] = jnp.zeros((16,), jnp.float32)
        return carry

    lax.fori_loop(0, CH, zfill, 0)
    for k in range(RPT // CH):
        pltpu.sync_copy(rows0, acc.at[pl.ds(base + k * CH, CH)])

    pltpu.sync_copy(dst_hbm.at[wid], idst)
    pltpu.sync_copy(src_hbm.at[wid, pl.ds(0, BLK)], isrc0)
    plsc.subcore_barrier()

    pltpu.async_copy(xs_hbm.at[isrc0.at[0]], rows0, sem0)

    def half_block(j0, cur, nxt, nxt_blk):
        pltpu.async_copy(src_hbm.at[wid, pl.ds(nxt_blk * BLK, BLK)], nxt, semi)
        for p in range(BLK // 2):
            a, b = 2 * p, 2 * p + 1
            pltpu.async_copy(xs_hbm.at[cur.at[b]], rows1, sem1)
            pltpu.make_async_copy(xs_hbm.at[cur.at[a]], rows0, sem0).wait()
            pltpu.sync_copy(rows0, acc.at[idst.at[j0 + a]], add=True)
            if b + 1 < BLK:
                pltpu.async_copy(xs_hbm.at[cur.at[b + 1]], rows0, sem0)
            else:
                pltpu.make_async_copy(
                    src_hbm.at[wid, pl.ds(0, BLK)], nxt, semi).wait()
                pltpu.async_copy(xs_hbm.at[nxt.at[0]], rows0, sem0)
            pltpu.make_async_copy(xs_hbm.at[cur.at[b]], rows1, sem1).wait()
            pltpu.sync_copy(rows1, acc.at[idst.at[j0 + b]], add=True)

    def body(t, carry):
        b0 = 2 * t
        half_block(b0 * BLK, isrc0, isrc1, b0 + 1)
        half_block(b0 * BLK + BLK, isrc1, isrc0, (b0 + 2) % NBLK)
        return carry

    lax.fori_loop(0, NBLK // 2, body, 0)
    pltpu.make_async_copy(xs_hbm.at[isrc0.at[0]], rows0, sem0).wait()
    plsc.subcore_barrier()

    def copy_out(k, carry):
        r0 = base + k * ZR
        pltpu.sync_copy(acc.at[pl.ds(r0, ZR)], out_hbm.at[c, pl.ds(r0, ZR)])
        return carry

    lax.fori_loop(0, NZ, copy_out, 0)


@functools.lru_cache(maxsize=None)
def _sc_kernels():
    mesh = plsc.VectorSubcoreMesh(core_axis_name="c", subcore_axis_name="s",
                                  num_cores=NC, num_subcores=NS)
    deg = pl.kernel(
        _deg_body,
        out_type=jax.ShapeDtypeStruct((NC, NR, 16), jnp.float32),
        mesh=mesh,
        compiler_params=pltpu.CompilerParams(use_tc_tiling_on_sc=False),
        scratch_types=[
            pltpu.VMEM((NCHUNK, CH), jnp.int32),
            pltpu.VMEM((CH, 16), jnp.float32),
            pltpu.VMEM((ZR, 16), jnp.float32),
            pltpu.VMEM_SHARED((NR, 16), jnp.float32),
        ],
    )
    agg = pl.kernel(
        _agg_body,
        out_type=jax.ShapeDtypeStruct((NC, NR, H), jnp.float32),
        mesh=mesh,
        scratch_types=[
            pltpu.VMEM((BLK, CH), jnp.int32),
            pltpu.VMEM((BLK, CH), jnp.int32),
            pltpu.VMEM((NCHUNK, CH), jnp.int32),
            pltpu.VMEM((CH, H), jnp.float32),
            pltpu.VMEM((CH, H), jnp.float32),
            pltpu.VMEM_SHARED((NR, H), jnp.float32),
            pltpu.SemaphoreType.DMA,
            pltpu.SemaphoreType.DMA,
            pltpu.SemaphoreType.DMA,
        ],
    )
    return deg, agg


def _tc0_body(x_ref, w1_ref, xw_ref):
    xw_ref[...] = jnp.dot(x_ref[...], w1_ref[...],
                          preferred_element_type=jnp.float32, precision=_PREC)


def _tc1_body(xw_ref, dpart_ref, xs_ref, dinv_ref):
    deg = dpart_ref[0, :N, :] + dpart_ref[1, :N, :] + 1.0
    dinv16 = lax.rsqrt(jnp.maximum(deg, 1.0))
    dinv = dinv16[:, 0:1]
    xs_ref[...] = xw_ref[...] * dinv
    dinv_ref[...] = dinv


def _tc2_body(a_ref, xs_ref, dinv_ref, b_ref, g_ref, be_ref, w_ref, out_ref):
    dinv = dinv_ref[...]
    h = (a_ref[0, :N, :] + a_ref[1, :N, :] + xs_ref[...]) * dinv + b_ref[...]
    mu = jnp.mean(h, axis=0, keepdims=True)
    var = jnp.mean((h - mu) ** 2, axis=0, keepdims=True)
    h = (h - mu) * lax.rsqrt(var + 1e-5) * g_ref[...] + be_ref[...]
    h = jnp.maximum(h, 0.0)
    out_ref[...] = jnp.dot(h, w_ref[...], preferred_element_type=jnp.float32,
                           precision=_PREC) * dinv


def _tc3_body(a_ref, xs_ref, dinv_ref, b_ref, g_ref, be_ref, wc_ref, bc_ref, out_ref):
    h = (a_ref[0, :N, :] + a_ref[1, :N, :] + xs_ref[...]) * dinv_ref[...] + b_ref[...]
    mu = jnp.mean(h, axis=0, keepdims=True)
    var = jnp.mean((h - mu) ** 2, axis=0, keepdims=True)
    h = (h - mu) * lax.rsqrt(var + 1e-5) * g_ref[...] + be_ref[...]
    out_ref[...] = jnp.dot(h, wc_ref[...], preferred_element_type=jnp.float32,
                           precision=_PREC) + bc_ref[...]


_tc0 = pl.pallas_call(
    _tc0_body,
    out_shape=jax.ShapeDtypeStruct((N, H), jnp.float32),
)

_tc1 = pl.pallas_call(
    _tc1_body,
    out_shape=[jax.ShapeDtypeStruct((N, H), jnp.float32),
               jax.ShapeDtypeStruct((N, 1), jnp.float32)],
)

_tc2 = pl.pallas_call(
    _tc2_body,
    out_shape=jax.ShapeDtypeStruct((N, H), jnp.float32),
)

_tc3 = pl.pallas_call(
    _tc3_body,
    out_shape=jax.ShapeDtypeStruct((N, OUT), jnp.float32),
)


def kernel(x, edge_index, W1, b1, g1, be1, W2, b2, g2, be2, Wc, bc):
    src = edge_index[0].astype(jnp.int32)
    dst = edge_index[1].astype(jnp.int32)
    pad_iota = jnp.arange(EPAD, dtype=jnp.int32)
    src3 = jnp.concatenate([src, pad_iota % N]).reshape(NW, NCHUNK, CH)
    dst3 = jnp.concatenate([dst, N + (pad_iota % PADROWS)]).reshape(NW, NCHUNK, CH)

    deg_kernel, agg_kernel = _sc_kernels()
    dparts = deg_kernel(dst3)
    xw1 = _tc0(x, W1)
    xs1, dinv = _tc1(xw1, dparts)
    agg1 = agg_kernel(xs1, src3, dst3)
    xs2 = _tc2(agg1, xs1, dinv, b1, g1, be1, W2)
    agg2 = agg_kernel(xs2, src3, dst3)
    out = _tc3(agg2, xs2, dinv, b2, g2, be2, Wc, bc)
    return out
